# baseline (speedup 1.0000x reference)
import functools
import math

import jax
import jax.numpy as jnp
from jax import lax
from jax.experimental import pallas as pl
from jax.experimental.pallas import tpu as pltpu
from jax.experimental.pallas import tpu_sc as plsc

D_MODEL = 64
SCALE = math.sqrt(D_MODEL)
NUM_CORES = 2
NUM_SUBCORES = 16
NW = NUM_CORES * NUM_SUBCORES
NBUF = 4
LA = 2


def _sc_embed(x, table2):
    S, T = x.shape
    rows_pw = S // NW
    n_groups = rows_pw
    half = T // 2
    mesh = plsc.VectorSubcoreMesh(
        core_axis_name="c", subcore_axis_name="s", num_cores=NUM_CORES
    )

    nchunk = (T + 15) // 16
    starts = [min(16 * c, T - 16) for c in range(nchunk)]

    @functools.partial(
        pl.kernel,
        out_type=jax.ShapeDtypeStruct((S * half, 2 * D_MODEL), jnp.float32),
        mesh=mesh,
        scratch_types=[
            pltpu.VMEM((rows_pw, T), jnp.int32),
            [pltpu.VMEM((T, 2 * D_MODEL), jnp.float32) for _ in range(NBUF)],
            [pltpu.VMEM((T,), jnp.int32) for _ in range(NBUF)],
            [pltpu.SemaphoreType.DMA for _ in range(NBUF)],
            [pltpu.SemaphoreType.DMA for _ in range(NBUF)],
        ],
        compiler_params=pltpu.CompilerParams(use_tc_tiling_on_sc=False),
    )
    def k(idx_hbm, table_hbm, out_hbm, idx_v, gbufs, qrows, gsems, wsems):
        wid = lax.axis_index("s") * NUM_CORES + lax.axis_index("c")
        row0 = wid * rows_pw
        orow0 = row0 * half
        pltpu.sync_copy(idx_hbm.at[pl.ds(row0, rows_pw)], idx_v)

        def prep(v, b):
            for st in starts:
                sl = pl.ds(st, 16)
                qrows[b][sl] = lax.shift_right_logical(idx_v[v, sl], 1)
            pltpu.async_copy(table_hbm.at[qrows[b]], gbufs[b], gsems[b])

        for v in range(LA):
            prep(v, v % NBUF)

        def visit(v, carry):
            @pl.when(v + LA < n_groups)
            def _ahead():
                for bb in range(NBUF):
                    @pl.when(lax.rem(v + LA, NBUF) == bb)
                    def _p():
                        @pl.when(v + LA >= NBUF)
                        def _wait_w():
                            pltpu.make_async_copy(
                                gbufs[bb].at[pl.ds(0, half)],
                                out_hbm.at[pl.ds(orow0, half)],
                                wsems[bb],
                            ).wait()
                        prep(v + LA, bb)

            for b in range(NBUF):
                @pl.when(lax.rem(v, NBUF) == b)
                def _work():
                    pltpu.make_async_copy(
                        table_hbm.at[qrows[b]], gbufs[b], gsems[b]
                    ).wait()

                    def blk_body(blk, c2):
                        base = lax.min(16 * blk, T - 16)
                        tb = lax.div(base, 2)
                        p64 = lax.mul(lax.rem(idx_v[v, pl.ds(base, 16)], 2), 64)
                        for kk in range(16):
                            off = p64[kk]
                            tr = tb + kk // 2
                            tc = (kk % 2) * 64
                            for c in range(D_MODEL // 16):
                                gbufs[b][tr, pl.ds(tc + 16 * c, 16)] = (
                                    gbufs[b][base + kk, pl.ds(off + 16 * c, 16)]
                                    * SCALE
                                )
                        return c2

                    lax.fori_loop(0, nchunk, blk_body, 0)

                    pltpu.async_copy(
                        gbufs[b].at[pl.ds(0, half)],
                        out_hbm.at[pl.ds(orow0 + v * half, half)],
                        wsems[b],
                    )
            return carry

        lax.fori_loop(0, n_groups, visit, 0)

        for b in range(min(NBUF, n_groups)):
            pltpu.make_async_copy(
                gbufs[b].at[pl.ds(0, half)],
                out_hbm.at[pl.ds(orow0, half)],
                wsems[b],
            ).wait()

    return k(x, table2)


def kernel(x, table):
    S, T = x.shape
    V = table.shape[0]
    table2 = table.reshape(V // 2, 2 * D_MODEL)
    out2 = _sc_embed(x.astype(jnp.int32), table2)
    return out2.reshape(S, T, D_MODEL)
